# Initial kernel scaffold; baseline (speedup 1.0000x reference)
import jax
import jax.numpy as jnp
from jax.experimental import pallas as pl


def kernel(head_embeddings, tail_embeddings, relation_table, relation_types):
    raise NotImplementedError("write your pallas kernel here")


# speedup vs baseline: 1.2752x; 1.2752x over previous
import functools

import jax
import jax.numpy as jnp
from jax import lax
from jax.experimental import pallas as pl
from jax.experimental.pallas import tpu as pltpu
from jax.experimental.pallas import tpu_sc as plsc

B = 16384
D = 128
L = 16
NC = 2
NS = 16
NW = NC * NS
B_PER_W = B // NW
CHUNK = 128
NCHUNKS = B_PER_W // CHUNK
GROUPS = CHUNK // L


def _body(head_hbm, tail_hbm, table_hbm, idx_hbm, out_hbm,
          idx_v, h_v, t_v, r_v, tr_v, sc_v, sem_h, sem_t, sem_r):
    wid = lax.axis_index("s") * NC + lax.axis_index("c")
    base = wid * B_PER_W

    lanes = lax.iota(jnp.int32, L)

    for c in range(NCHUNKS):
        cbase = base + c * CHUNK
        pltpu.sync_copy(idx_hbm.at[pl.ds(cbase, CHUNK)], idx_v)
        cp_r = pltpu.async_copy(table_hbm.at[idx_v], r_v, sem_r)
        cp_h = pltpu.async_copy(head_hbm.at[pl.ds(cbase, CHUNK), :], h_v, sem_h)
        cp_t = pltpu.async_copy(tail_hbm.at[pl.ds(cbase, CHUNK), :], t_v, sem_t)
        cp_r.wait()
        cp_h.wait()
        cp_t.wait()

        def group(g, _):
            for r in range(L):
                row = g * L + r
                sl = pl.ds(0, L)
                acc = h_v[row, sl] * r_v[row, sl] * t_v[row, sl]
                for k in range(1, D // L):
                    sl = pl.ds(k * L, L)
                    acc = acc + h_v[row, sl] * r_v[row, sl] * t_v[row, sl]
                tr_v[r, :] = acc
            s = plsc.load_gather(tr_v, [lanes, jnp.zeros((L,), jnp.int32)])
            for j in range(1, L):
                s = s + plsc.load_gather(
                    tr_v, [lanes, jnp.full((L,), j, jnp.int32)])
            sc_v[pl.ds(g * L, L)] = s
            return ()

        lax.fori_loop(0, GROUPS, group, (), unroll=False)
        pltpu.sync_copy(sc_v, out_hbm.at[pl.ds(cbase, CHUNK)])


@jax.jit
def _distmult_sc(head, tail, table, idx):
    mesh = plsc.VectorSubcoreMesh(core_axis_name="c", subcore_axis_name="s")
    kern = pl.kernel(
        _body,
        out_type=jax.ShapeDtypeStruct((B,), jnp.float32),
        mesh=mesh,
        compiler_params=pltpu.CompilerParams(needs_layout_passes=False),
        scratch_types=[
            pltpu.VMEM((CHUNK,), jnp.int32),
            pltpu.VMEM((CHUNK, D), jnp.float32),
            pltpu.VMEM((CHUNK, D), jnp.float32),
            pltpu.VMEM((CHUNK, D), jnp.float32),
            pltpu.VMEM((L, L), jnp.float32),
            pltpu.VMEM((CHUNK,), jnp.float32),
            pltpu.SemaphoreType.DMA,
            pltpu.SemaphoreType.DMA,
            pltpu.SemaphoreType.DMA,
        ],
    )
    return kern(head, tail, table, idx)


def kernel(head_embeddings, tail_embeddings, relation_table, relation_types):
    idx = relation_types.astype(jnp.int32)
    return _distmult_sc(head_embeddings, tail_embeddings, relation_table, idx)
